# baseline (speedup 1.0000x reference)
import functools

import jax
import jax.numpy as jnp
from jax import lax
from jax.experimental import pallas as pl
from jax.experimental.pallas import tpu as pltpu
from jax.experimental.pallas import tpu_sc as plsc

VOCAB = 100000
DIM = 64
B = 4096
L = 50

TOTAL = B * L
NC = 2
NS = 16
NW = NC * NS
PER_W = TOTAL // NW
CHUNK = 128
NCH = PER_W // CHUNK
K = 5
NG = NCH // K

TBL_BLK = 4096
TBL_GRID = -(-VOCAB // TBL_BLK)


def _transform_body(tt_ref, w_ref, b_ref, out_ref):
    x = lax.dot_general(
        tt_ref[...], w_ref[...],
        dimension_numbers=(((0,), (1,)), ((), ())),
        preferred_element_type=jnp.float32,
    )
    y = jax.nn.sigmoid(x + b_ref[...])
    out_ref[...] = jnp.concatenate([y, y], axis=1)


def _transform_table(tt, W, b2d):
    return pl.pallas_call(
        _transform_body,
        grid=(TBL_GRID,),
        in_specs=[
            pl.BlockSpec((DIM, TBL_BLK), lambda i: (0, i)),
            pl.BlockSpec((DIM, DIM), lambda i: (0, 0)),
            pl.BlockSpec((1, DIM), lambda i: (0, 0)),
        ],
        out_specs=pl.BlockSpec((TBL_BLK, 2 * DIM), lambda i: (i, 0)),
        out_shape=jax.ShapeDtypeStruct((TBL_GRID * TBL_BLK, 2 * DIM), jnp.float32),
    )(tt, W, b2d)


_sc_mesh = plsc.VectorSubcoreMesh(core_axis_name="c", subcore_axis_name="s")


CPL = B // CHUNK // 2


def _idx_body(bt_ref, out_ref):
    out_ref[...] = (bt_ref[...] * 2).reshape(L * B // CHUNK, CHUNK)


def _make_idx(bt):
    return pl.pallas_call(
        _idx_body,
        in_specs=[pl.BlockSpec((L, B), lambda: (0, 0))],
        out_specs=pl.BlockSpec((L * B // CHUNK, CHUNK), lambda: (0, 0)),
        out_shape=jax.ShapeDtypeStruct((L * B // CHUNK, CHUNK), jnp.int32),
    )(bt)


LH = L // 2
NCH_H = NCH // 2


def _make_sc_gather(half):
    @functools.partial(
        pl.kernel,
        out_type=jax.ShapeDtypeStruct((LH * B // 2, 2 * DIM), jnp.float32),
        mesh=_sc_mesh,
        scratch_types=[
            pltpu.VMEM((NCH_H, CHUNK), jnp.int32),
            pltpu.VMEM((2, K * CHUNK, DIM), jnp.float32),
            pltpu.SemaphoreType.DMA,
            pltpu.SemaphoreType.DMA,
        ],
        compiler_params=pltpu.CompilerParams(use_tc_tiling_on_sc=False),
    )
    def _sc_gather(tprime_hbm, idx_hbm, out_hbm, idx_v, rows_v, sem0, sem1):
        wid = lax.axis_index("s") * NC + lax.axis_index("c")
        cbase = half * (NW * NCH_H) + wid * NCH_H
        pltpu.sync_copy(idx_hbm.at[pl.ds(cbase, NCH_H)], idx_v)

        def fire(g, buf, sem):
            return [
                pltpu.async_copy(
                    tprime_hbm.at[idx_v.at[g * K + j]],
                    rows_v.at[buf].at[pl.ds(j * CHUNK, CHUNK)], sem)
                for j in range(K)
            ]

        def writeback(g, buf):
            for j in range(K):
                c = cbase + g * K + j
                ci = lax.rem(c, 32)
                mrow = (lax.div(c, 32) - half * LH) * (B // 2) \
                    + lax.rem(ci, CPL) * CHUNK
                par = lax.div(ci, CPL)
                pltpu.sync_copy(
                    rows_v.at[buf].at[pl.ds(j * CHUNK, CHUNK)],
                    out_hbm.at[pl.ds(mrow, CHUNK), pl.ds(par * DIM, DIM)])

        @pl.loop(0, 4, step=2)
        def _groups(e):
            h0 = fire(e, 0, sem0)
            h1 = fire(e + 1, 1, sem1)
            for h in h0:
                h.wait()
            writeback(e, 0)
            for h in h1:
                h.wait()
            writeback(e + 1, 1)

        h4 = fire(4, 0, sem0)
        for h in h4:
            h.wait()
        writeback(4, 0)

    return _sc_gather


_sc_gather_a = _make_sc_gather(0)
_sc_gather_b = _make_sc_gather(1)


NB = 4
NBLK = B // NB


def _transpose_slot(x):
    xt = jnp.transpose(x)
    return jnp.concatenate([xt[0:DIM, :], xt[DIM:2 * DIM, :]], axis=1)


def _transpose_body_a(g_ref, out_ref):
    out_ref[...] = _transpose_slot(g_ref[...])[None]


def _transpose_body_b(buf_ref, g_ref, out_ref):
    del buf_ref
    out_ref[...] = _transpose_slot(g_ref[...])[None]


def _transpose_a(g128):
    return pl.pallas_call(
        _transpose_body_a,
        grid=(LH,),
        in_specs=[pl.BlockSpec((B // 2, 2 * DIM), lambda l: (l, 0))],
        out_specs=pl.BlockSpec((1, DIM, B), lambda l: (l, 0, 0)),
        out_shape=jax.ShapeDtypeStruct((L, DIM, B), jnp.float32),
    )(g128)


def _transpose_b(buf, g128):
    return pl.pallas_call(
        _transpose_body_b,
        grid=(LH,),
        in_specs=[
            pl.BlockSpec(memory_space=pl.ANY),
            pl.BlockSpec((B // 2, 2 * DIM), lambda l: (l, 0)),
        ],
        out_specs=pl.BlockSpec((1, DIM, B), lambda l: (l + LH, 0, 0)),
        out_shape=jax.ShapeDtypeStruct((L, DIM, B), jnp.float32),
        input_output_aliases={0: 0},
    )(buf, g128)


def kernel(batch, table, W, b):
    tt = jnp.transpose(table)
    t128 = _transform_table(tt, W, b.reshape(1, DIM))
    tlin = t128.reshape(TBL_GRID * TBL_BLK * 2, DIM)
    bt = jnp.transpose(batch).astype(jnp.int32)
    idx = _make_idx(bt)
    g_a = _sc_gather_a(tlin, idx)
    g_b = _sc_gather_b(tlin, idx)
    buf = _transpose_a(g_a)
    out3 = _transpose_b(buf, g_b)
    return jnp.transpose(out3, (2, 0, 1))


# speedup vs baseline: 1.0787x; 1.0787x over previous
import functools

import jax
import jax.numpy as jnp
from jax import lax
from jax.experimental import pallas as pl
from jax.experimental.pallas import tpu as pltpu
from jax.experimental.pallas import tpu_sc as plsc

VOCAB = 100000
DIM = 64
B = 4096
L = 50

TOTAL = B * L
NC = 2
NS = 16
NW = NC * NS
PER_W = TOTAL // NW
CHUNK = 128
NCH = PER_W // CHUNK
K = 5
NG = NCH // K

TBL_BLK = 8192
TBL_GRID = -(-VOCAB // TBL_BLK)


def _transform_body(tt_ref, w_ref, b_ref, out_ref):
    x = lax.dot_general(
        tt_ref[...], w_ref[...],
        dimension_numbers=(((0,), (1,)), ((), ())),
        preferred_element_type=jnp.float32,
    )
    y = jax.nn.sigmoid(x + b_ref[...])
    out_ref[...] = jnp.concatenate([y, y], axis=1)


def _transform_table(tt, W, b2d):
    return pl.pallas_call(
        _transform_body,
        grid=(TBL_GRID,),
        in_specs=[
            pl.BlockSpec((DIM, TBL_BLK), lambda i: (0, i)),
            pl.BlockSpec((DIM, DIM), lambda i: (0, 0)),
            pl.BlockSpec((1, DIM), lambda i: (0, 0)),
        ],
        out_specs=pl.BlockSpec((TBL_BLK, 2 * DIM), lambda i: (i, 0)),
        out_shape=jax.ShapeDtypeStruct((TBL_GRID * TBL_BLK, 2 * DIM), jnp.float32),
    )(tt, W, b2d)


_sc_mesh = plsc.VectorSubcoreMesh(core_axis_name="c", subcore_axis_name="s")


CPL = B // CHUNK // 2


def _idx_body(bt_ref, out_ref):
    out_ref[...] = (bt_ref[...] * 2).reshape(L * B // CHUNK, CHUNK)


def _make_idx(bt):
    return pl.pallas_call(
        _idx_body,
        in_specs=[pl.BlockSpec((L, B), lambda: (0, 0))],
        out_specs=pl.BlockSpec((L * B // CHUNK, CHUNK), lambda: (0, 0)),
        out_shape=jax.ShapeDtypeStruct((L * B // CHUNK, CHUNK), jnp.int32),
    )(bt)


@functools.partial(
    pl.kernel,
    out_type=jax.ShapeDtypeStruct((TOTAL // 2, 2 * DIM), jnp.float32),
    mesh=_sc_mesh,
    scratch_types=[
        pltpu.VMEM((NCH, CHUNK), jnp.int32),
        pltpu.VMEM((2, K * CHUNK, DIM), jnp.float32),
        pltpu.SemaphoreType.DMA,
        pltpu.SemaphoreType.DMA,
    ],
    compiler_params=pltpu.CompilerParams(use_tc_tiling_on_sc=False),
)
def _sc_gather(tprime_hbm, idx_hbm, out_hbm, idx_v, rows_v, sem0, sem1):
    wid = lax.axis_index("s") * NC + lax.axis_index("c")
    cbase = wid * NCH
    pltpu.sync_copy(idx_hbm.at[pl.ds(cbase, NCH)], idx_v)

    def fire(g, buf, sem):
        return [
            pltpu.async_copy(
                tprime_hbm.at[idx_v.at[g * K + j]],
                rows_v.at[buf].at[pl.ds(j * CHUNK, CHUNK)], sem)
            for j in range(K)
        ]

    def writeback(g, buf):
        for j in range(K):
            c = cbase + g * K + j
            ci = lax.rem(c, 32)
            mrow = lax.div(c, 32) * (B // 2) + lax.rem(ci, CPL) * CHUNK
            par = lax.div(ci, CPL)
            pltpu.sync_copy(
                rows_v.at[buf].at[pl.ds(j * CHUNK, CHUNK)],
                out_hbm.at[pl.ds(mrow, CHUNK), pl.ds(par * DIM, DIM)])

    @pl.loop(0, NG, step=2)
    def _groups(e):
        h0 = fire(e, 0, sem0)
        h1 = fire(e + 1, 1, sem1)
        for h in h0:
            h.wait()
        writeback(e, 0)
        for h in h1:
            h.wait()
        writeback(e + 1, 1)


NB = 4
NBLK = B // NB


def _transpose_body(g_ref, out_ref):
    for s in range(2):
        x = g_ref[pl.ds(s * (B // 2), B // 2), :]
        xt = jnp.transpose(x)
        out_ref[s] = jnp.concatenate([xt[0:DIM, :], xt[DIM:2 * DIM, :]],
                                     axis=1)


def _transpose_out(g128):
    return pl.pallas_call(
        _transpose_body,
        grid=(L // 2,),
        in_specs=[
            pl.BlockSpec((B, 2 * DIM), lambda l: (l, 0)),
        ],
        out_specs=pl.BlockSpec((2, DIM, B), lambda l: (l, 0, 0)),
        out_shape=jax.ShapeDtypeStruct((L, DIM, B), jnp.float32),
    )(g128)


def kernel(batch, table, W, b):
    tt = jnp.transpose(table)
    t128 = _transform_table(tt, W, b.reshape(1, DIM))
    tlin = t128.reshape(TBL_GRID * TBL_BLK * 2, DIM)
    bt = jnp.transpose(batch).astype(jnp.int32)
    idx = _make_idx(bt)
    gathered = _sc_gather(tlin, idx)
    out3 = _transpose_out(gathered)
    return jnp.transpose(out3, (2, 0, 1))


# speedup vs baseline: 1.0808x; 1.0020x over previous
import functools

import jax
import jax.numpy as jnp
from jax import lax
from jax.experimental import pallas as pl
from jax.experimental.pallas import tpu as pltpu
from jax.experimental.pallas import tpu_sc as plsc

VOCAB = 100000
DIM = 64
B = 4096
L = 50

TOTAL = B * L
NC = 2
NS = 16
NW = NC * NS
PER_W = TOTAL // NW
CHUNK = 128
NCH = PER_W // CHUNK
K = 5
NG = NCH // K

TBL_BLK = 16384
TBL_GRID = -(-VOCAB // TBL_BLK)


def _transform_body(tt_ref, w_ref, b_ref, out_ref):
    x = lax.dot_general(
        tt_ref[...], w_ref[...],
        dimension_numbers=(((0,), (1,)), ((), ())),
        preferred_element_type=jnp.float32,
    )
    y = jax.nn.sigmoid(x + b_ref[...])
    out_ref[...] = jnp.concatenate([y, y], axis=1)


def _transform_table(tt, W, b2d):
    return pl.pallas_call(
        _transform_body,
        grid=(TBL_GRID,),
        in_specs=[
            pl.BlockSpec((DIM, TBL_BLK), lambda i: (0, i)),
            pl.BlockSpec((DIM, DIM), lambda i: (0, 0)),
            pl.BlockSpec((1, DIM), lambda i: (0, 0)),
        ],
        out_specs=pl.BlockSpec((TBL_BLK, 2 * DIM), lambda i: (i, 0)),
        out_shape=jax.ShapeDtypeStruct((TBL_GRID * TBL_BLK, 2 * DIM), jnp.float32),
    )(tt, W, b2d)


_sc_mesh = plsc.VectorSubcoreMesh(core_axis_name="c", subcore_axis_name="s")


CPL = B // CHUNK // 2


def _idx_body(bt_ref, out_ref):
    out_ref[...] = (bt_ref[...] * 2).reshape(L * B // CHUNK, CHUNK)


def _make_idx(bt):
    return pl.pallas_call(
        _idx_body,
        in_specs=[pl.BlockSpec((L, B), lambda: (0, 0))],
        out_specs=pl.BlockSpec((L * B // CHUNK, CHUNK), lambda: (0, 0)),
        out_shape=jax.ShapeDtypeStruct((L * B // CHUNK, CHUNK), jnp.int32),
    )(bt)


@functools.partial(
    pl.kernel,
    out_type=jax.ShapeDtypeStruct((TOTAL // 2, 2 * DIM), jnp.float32),
    mesh=_sc_mesh,
    scratch_types=[
        pltpu.VMEM((NCH, CHUNK), jnp.int32),
        pltpu.VMEM((2, K * CHUNK, DIM), jnp.float32),
        pltpu.SemaphoreType.DMA,
        pltpu.SemaphoreType.DMA,
    ],
    compiler_params=pltpu.CompilerParams(use_tc_tiling_on_sc=False),
)
def _sc_gather(tprime_hbm, idx_hbm, out_hbm, idx_v, rows_v, sem0, sem1):
    wid = lax.axis_index("s") * NC + lax.axis_index("c")
    cbase = wid * NCH
    pltpu.sync_copy(idx_hbm.at[pl.ds(cbase, NCH)], idx_v)

    def fire(g, buf, sem):
        return [
            pltpu.async_copy(
                tprime_hbm.at[idx_v.at[g * K + j]],
                rows_v.at[buf].at[pl.ds(j * CHUNK, CHUNK)], sem)
            for j in range(K)
        ]

    def writeback(g, buf):
        for j in range(K):
            c = cbase + g * K + j
            ci = lax.rem(c, 32)
            mrow = lax.div(c, 32) * (B // 2) + lax.rem(ci, CPL) * CHUNK
            par = lax.div(ci, CPL)
            pltpu.sync_copy(
                rows_v.at[buf].at[pl.ds(j * CHUNK, CHUNK)],
                out_hbm.at[pl.ds(mrow, CHUNK), pl.ds(par * DIM, DIM)])

    @pl.loop(0, NG, step=2)
    def _groups(e):
        h0 = fire(e, 0, sem0)
        h1 = fire(e + 1, 1, sem1)
        for h in h0:
            h.wait()
        writeback(e, 0)
        for h in h1:
            h.wait()
        writeback(e + 1, 1)


NB = 4
NBLK = B // NB


def _transpose_body(g_ref, out_ref):
    for s in range(2):
        x = g_ref[pl.ds(s * (B // 2), B // 2), :]
        xt = jnp.transpose(x)
        out_ref[s] = jnp.concatenate([xt[0:DIM, :], xt[DIM:2 * DIM, :]],
                                     axis=1)


def _transpose_out(g128):
    return pl.pallas_call(
        _transpose_body,
        grid=(L // 2,),
        in_specs=[
            pl.BlockSpec((B, 2 * DIM), lambda l: (l, 0)),
        ],
        out_specs=pl.BlockSpec((2, DIM, B), lambda l: (l, 0, 0)),
        out_shape=jax.ShapeDtypeStruct((L, DIM, B), jnp.float32),
    )(g128)


def kernel(batch, table, W, b):
    tt = jnp.transpose(table)
    t128 = _transform_table(tt, W, b.reshape(1, DIM))
    tlin = t128.reshape(TBL_GRID * TBL_BLK * 2, DIM)
    bt = jnp.transpose(batch).astype(jnp.int32)
    idx = _make_idx(bt)
    gathered = _sc_gather(tlin, idx)
    out3 = _transpose_out(gathered)
    return jnp.transpose(out3, (2, 0, 1))
